# baseline (speedup 1.0000x reference)
import jax
import jax.numpy as jnp
from jax.experimental import pallas as pl
from jax.experimental.pallas import tpu as pltpu

B_ = 1024
S_ = 50
NUM_CLASSES_ = 1000
K_ = 6


def _onehot_body(x_ref, o_hbm, scratch, sems):
    ids = jax.lax.broadcasted_iota(jnp.int32, (1, NUM_CLASSES_, B_), 1)

    def copy(i):
        slot = i % K_
        return pltpu.make_async_copy(
            scratch.at[slot],
            o_hbm.at[i],
            sems.at[slot],
        )

    for i in range(S_):
        if i >= K_:
            copy(i - K_).wait()
        xv = x_ref[pl.ds(i, 1)]
        scratch[i % K_] = (ids == xv).astype(scratch.dtype)[0]
        copy(i).start()

    for i in range(S_ - K_, S_):
        copy(i).wait()


def kernel(x):
    out_dtype = jnp.zeros((), jnp.int64).dtype
    xt = jnp.transpose(x).astype(jnp.int32).reshape(S_, 1, B_)
    out_t = pl.pallas_call(
        _onehot_body,
        in_specs=[pl.BlockSpec(memory_space=pltpu.MemorySpace.VMEM)],
        out_specs=pl.BlockSpec(memory_space=pltpu.MemorySpace.HBM),
        out_shape=jax.ShapeDtypeStruct((S_, NUM_CLASSES_, B_), out_dtype),
        scratch_shapes=[
            pltpu.MemorySpace.VMEM((K_, NUM_CLASSES_, B_), jnp.int32),
            pltpu.SemaphoreType.DMA((K_,)),
        ],
        compiler_params=pltpu.CompilerParams(
            vmem_limit_bytes=100 * 1024 * 1024,
        ),
    )(xt)
    return jnp.transpose(out_t, (2, 0, 1))


# speedup vs baseline: 1.0134x; 1.0134x over previous
import jax
import jax.numpy as jnp
from jax.experimental import pallas as pl
from jax.experimental.pallas import tpu as pltpu

B_ = 1024
S_ = 50
NUM_CLASSES_ = 1000
SBLK_ = 2


def _onehot_block(x_ref, o_ref):
    ids = jax.lax.broadcasted_iota(jnp.int32, o_ref.shape, 1)
    o_ref[...] = (ids == x_ref[...]).astype(o_ref.dtype)


def kernel(x):
    out_dtype = jnp.zeros((), jnp.int64).dtype
    xt = jnp.transpose(x).astype(jnp.int32).reshape(S_, 1, B_)
    out_t = pl.pallas_call(
        _onehot_block,
        grid=(S_ // SBLK_,),
        in_specs=[pl.BlockSpec((SBLK_, 1, B_), lambda i: (i, 0, 0))],
        out_specs=pl.BlockSpec((SBLK_, NUM_CLASSES_, B_), lambda i: (i, 0, 0)),
        out_shape=jax.ShapeDtypeStruct((S_, NUM_CLASSES_, B_), out_dtype),
    )(xt)
    return jnp.transpose(out_t, (2, 0, 1))
